# baseline (speedup 1.0000x reference)
import functools

import jax
import jax.numpy as jnp
from jax import lax
from jax.experimental import pallas as pl
from jax.experimental.pallas import tpu as pltpu
from jax.experimental.pallas import tpu_sc as plsc

NC = 2
NS = 16
NW = NC * NS
LK = 128
BN = 1024


@functools.cache
def _make_sc_agg(n_pad: int, h_dim: int, cha: int, chb: int):
    rpt = n_pad // NS

    mesh = plsc.VectorSubcoreMesh(core_axis_name="c", subcore_axis_name="s",
                                  num_cores=NC, num_subcores=NS)

    @functools.partial(
        pl.kernel,
        out_type=jax.ShapeDtypeStruct((NC * n_pad, h_dim), jnp.float32),
        mesh=mesh,
        scratch_types=[
            pltpu.VMEM((cha, LK), jnp.int32),
            pltpu.VMEM((cha, LK), jnp.int32),
            pltpu.VMEM((LK, h_dim), jnp.float32),
            pltpu.VMEM_SHARED((n_pad, h_dim), jnp.float32),
            pltpu.SemaphoreType.DMA,
        ],
    )
    def sc_agg(hr_hbm, srca_hbm, dsta_hbm, srcb_hbm, dstb_hbm, out_hbm,
               src_v, dst_v, rows0_v, agg_sh, sem0):
        c = lax.axis_index("c")
        s = lax.axis_index("s")
        w = c * NS + s

        zvec = jnp.zeros((16,), jnp.float32)

        def zrow(i, carry):
            for j in range(h_dim // 16):
                rows0_v[i, pl.ds(j * 16, 16)] = zvec
            return carry

        lax.fori_loop(0, LK, zrow, 0)

        def zcopy(k, carry):
            pltpu.sync_copy(rows0_v, agg_sh.at[pl.ds(s * rpt + k * LK, LK)])
            return carry

        lax.fori_loop(0, rpt // LK, zcopy, 0)

        plsc.subcore_barrier()

        def body(j, carry):
            pltpu.async_copy(hr_hbm.at[src_v.at[j]], rows0_v, sem0).wait()
            pltpu.sync_copy(rows0_v, agg_sh.at[dst_v.at[j]], add=True)
            return carry

        pltpu.sync_copy(srca_hbm.at[w], src_v)
        pltpu.sync_copy(dsta_hbm.at[w], dst_v)
        lax.fori_loop(0, cha, body, 0)

        @pl.when(c == 0)
        def _extra():
            pltpu.sync_copy(srcb_hbm.at[s], src_v.at[pl.ds(0, chb)])
            pltpu.sync_copy(dstb_hbm.at[s], dst_v.at[pl.ds(0, chb)])
            lax.fori_loop(0, chb, body, 0)

        plsc.subcore_barrier()

        pltpu.sync_copy(agg_sh.at[pl.ds(s * rpt, rpt)],
                        out_hbm.at[pl.ds(c * n_pad + s * rpt, rpt)])

    return sc_agg


def _tc_proj(h, wrel, wroot, b):
    n_pad, d = h.shape
    hd = wrel.shape[1]
    grid = n_pad // BN

    def body(h_ref, wrel_ref, wroot_ref, b_ref, hr_ref, hroot_ref):
        hb = h_ref[...]
        hr_ref[...] = jnp.dot(hb, wrel_ref[...],
                              preferred_element_type=jnp.float32)
        hroot_ref[...] = jnp.dot(hb, wroot_ref[...],
                                 preferred_element_type=jnp.float32) + b_ref[...]

    return pl.pallas_call(
        body,
        grid=(grid,),
        in_specs=[
            pl.BlockSpec((BN, d), lambda i: (i, 0)),
            pl.BlockSpec((d, hd), lambda i: (0, 0)),
            pl.BlockSpec((d, hd), lambda i: (0, 0)),
            pl.BlockSpec((1, hd), lambda i: (0, 0)),
        ],
        out_specs=[
            pl.BlockSpec((BN, hd), lambda i: (i, 0)),
            pl.BlockSpec((BN, hd), lambda i: (i, 0)),
        ],
        out_shape=[jax.ShapeDtypeStruct((n_pad, hd), jnp.float32)] * 2,
    )(h, wrel, wroot, b.reshape(1, hd))


def _tc_combine_proj(agg, root_prev, wrel, wroot, b):
    n2, hd = agg.shape
    n_pad = n2 // NC
    grid = n_pad // BN

    def body(a0_ref, a1_ref, root_ref, wrel_ref, wroot_ref, b_ref,
             hr_ref, hroot_ref):
        hb = jnp.maximum(a0_ref[...] + a1_ref[...] + root_ref[...], 0.0)
        hr_ref[...] = jnp.dot(hb, wrel_ref[...],
                              preferred_element_type=jnp.float32)
        hroot_ref[...] = jnp.dot(hb, wroot_ref[...],
                                 preferred_element_type=jnp.float32) + b_ref[...]

    return pl.pallas_call(
        body,
        grid=(grid,),
        in_specs=[
            pl.BlockSpec((BN, hd), lambda i: (i, 0)),
            pl.BlockSpec((BN, hd), lambda i: (i + grid, 0)),
            pl.BlockSpec((BN, hd), lambda i: (i, 0)),
            pl.BlockSpec((hd, hd), lambda i: (0, 0)),
            pl.BlockSpec((hd, hd), lambda i: (0, 0)),
            pl.BlockSpec((1, hd), lambda i: (0, 0)),
        ],
        out_specs=[
            pl.BlockSpec((BN, hd), lambda i: (i, 0)),
            pl.BlockSpec((BN, hd), lambda i: (i, 0)),
        ],
        out_shape=[jax.ShapeDtypeStruct((n_pad, hd), jnp.float32)] * 2,
    )(agg, agg, root_prev, wrel, wroot, b.reshape(1, hd))


def _tc_final(agg, root_prev, batch3d, wl, bl, g: int):
    n2, hd = agg.shape
    n_pad = n2 // NC
    grid = n_pad // BN
    c_dim = wl.shape[1]

    def body(a0_ref, a1_ref, root_ref, batch_ref, wl_ref, bl_ref,
             pooled_ref, out_ref, sums_ref, cnt_ref):
        i = pl.program_id(0)

        @pl.when(i == 0)
        def _init():
            sums_ref[...] = jnp.zeros_like(sums_ref)
            cnt_ref[...] = jnp.zeros_like(cnt_ref)

        h3 = a0_ref[...] + a1_ref[...] + root_ref[...]
        bvec = batch_ref[0]
        pt = (lax.broadcasted_iota(jnp.int32, (g, 1), 0) == bvec
              ).astype(jnp.float32)
        sums_ref[...] += jnp.dot(pt, h3, preferred_element_type=jnp.float32)
        cnt_ref[...] += jnp.broadcast_to(
            jnp.sum(pt, axis=1, keepdims=True), cnt_ref.shape)

        @pl.when(i == grid - 1)
        def _fin():
            pooled = sums_ref[...] / jnp.maximum(cnt_ref[...], 1.0)
            pooled_ref[...] = pooled
            out_ref[...] = jnp.dot(pooled, wl_ref[...],
                                   preferred_element_type=jnp.float32) + bl_ref[...]

    return pl.pallas_call(
        body,
        grid=(grid,),
        in_specs=[
            pl.BlockSpec((BN, hd), lambda i: (i, 0)),
            pl.BlockSpec((BN, hd), lambda i: (i + grid, 0)),
            pl.BlockSpec((BN, hd), lambda i: (i, 0)),
            pl.BlockSpec((1, 1, BN), lambda i: (i, 0, 0)),
            pl.BlockSpec((hd, c_dim), lambda i: (0, 0)),
            pl.BlockSpec((1, c_dim), lambda i: (0, 0)),
        ],
        out_specs=[
            pl.BlockSpec((g, hd), lambda i: (0, 0)),
            pl.BlockSpec((g, c_dim), lambda i: (0, 0)),
        ],
        out_shape=[
            jax.ShapeDtypeStruct((g, hd), jnp.float32),
            jax.ShapeDtypeStruct((g, c_dim), jnp.float32),
        ],
        scratch_shapes=[
            pltpu.VMEM((g, hd), jnp.float32),
            pltpu.VMEM((g, hd), jnp.float32),
        ],
    )(agg, agg, root_prev, batch3d, wl, bl.reshape(1, c_dim))


def kernel(x, edge_index, batch, W1_rel, W1_root, b1, W2_rel, W2_root, b2,
           W3_rel, W3_root, b3, Wl, bl):
    n, d = x.shape
    h_dim = W1_rel.shape[1]
    e = edge_index.shape[1]
    g = 64

    n_pad = ((n + BN) // BN) * BN
    tot_needed = -(-e // LK)
    cha = max(8, 8 * (tot_needed // (44 * 8)))
    chb = max(8, 8 * (-(-(tot_needed - 2 * NS * cha) // (NS * 8))))
    e_pad = (2 * NS * cha + NS * chb) * LK

    x_p = jnp.zeros((n_pad, d), x.dtype).at[:n].set(x)
    pad_e = jnp.full((e_pad - e,), n, jnp.int32)
    edges_flat_s = jnp.concatenate([edge_index[0], pad_e])
    edges_flat_d = jnp.concatenate([edge_index[1], pad_e])
    cut = 2 * NS * cha * LK
    src_a = edges_flat_s[:cut].reshape(NW, cha, LK)
    dst_a = edges_flat_d[:cut].reshape(NW, cha, LK)
    src_b = edges_flat_s[cut:].reshape(NS, chb, LK)
    dst_b = edges_flat_d[cut:].reshape(NS, chb, LK)
    batch_p = jnp.concatenate(
        [batch.astype(jnp.int32), jnp.full((n_pad - n,), g, jnp.int32)]
    ).reshape(n_pad // BN, 1, BN)

    sc_agg = _make_sc_agg(n_pad, h_dim, cha, chb)

    xr, xroot = _tc_proj(x_p, W1_rel, W1_root, b1)
    agg1 = sc_agg(xr, src_a, dst_a, src_b, dst_b)
    h1r, h1root = _tc_combine_proj(agg1, xroot, W2_rel, W2_root, b2)
    agg2 = sc_agg(h1r, src_a, dst_a, src_b, dst_b)
    h2r, h2root = _tc_combine_proj(agg2, h1root, W3_rel, W3_root, b3)
    agg3 = sc_agg(h2r, src_a, dst_a, src_b, dst_b)
    pooled, out = _tc_final(agg3, h2root, batch_p, Wl, bl, g)
    return (pooled, out)


# speedup vs baseline: 1.8052x; 1.8052x over previous
import functools

import jax
import jax.numpy as jnp
from jax import lax
from jax.experimental import pallas as pl
from jax.experimental.pallas import tpu as pltpu
from jax.experimental.pallas import tpu_sc as plsc

NC = 2
NS = 16
NW = NC * NS
LK = 128
BN = 1024


@functools.cache
def _make_sc_agg(n_pad: int, h_dim: int, ch: int):
    rpt = n_pad // NS
    zr = 64

    mesh = plsc.VectorSubcoreMesh(core_axis_name="c", subcore_axis_name="s",
                                  num_cores=NC, num_subcores=NS)

    @functools.partial(
        pl.kernel,
        out_type=jax.ShapeDtypeStruct((NC * n_pad, h_dim), jnp.float32),
        mesh=mesh,
        scratch_types=[
            pltpu.VMEM((ch, LK), jnp.int32),
            pltpu.VMEM((ch, LK), jnp.int32),
            pltpu.VMEM((LK, h_dim), jnp.float32),
            pltpu.VMEM((zr, h_dim), jnp.float32),
            pltpu.VMEM_SHARED((n_pad, h_dim), jnp.float32),
            pltpu.SemaphoreType.DMA,
        ],
    )
    def sc_agg(hr_hbm, src_hbm, dst_hbm, out_hbm,
               src_v, dst_v, rows_v, zero_v, agg_sh, sem):
        c = lax.axis_index("c")
        s = lax.axis_index("s")
        w = c * NS + s

        zvec = jnp.zeros((16,), jnp.float32)

        def zrow(i, carry):
            for j in range(h_dim // 16):
                zero_v[i, pl.ds(j * 16, 16)] = zvec
            return carry

        lax.fori_loop(0, zr, zrow, 0)

        def zcopy(k, carry):
            pltpu.sync_copy(zero_v, agg_sh.at[pl.ds(s * rpt + k * zr, zr)])
            return carry

        lax.fori_loop(0, rpt // zr, zcopy, 0)

        pltpu.sync_copy(src_hbm.at[w], src_v)
        pltpu.sync_copy(dst_hbm.at[w], dst_v)
        plsc.subcore_barrier()

        def body(j, carry):
            pltpu.async_copy(hr_hbm.at[src_v.at[j]], rows_v, sem).wait()
            pltpu.sync_copy(rows_v, agg_sh.at[dst_v.at[j]], add=True)
            return carry

        lax.fori_loop(0, ch, body, 0)
        plsc.subcore_barrier()

        pltpu.sync_copy(agg_sh.at[pl.ds(s * rpt, rpt)],
                        out_hbm.at[pl.ds(c * n_pad + s * rpt, rpt)])

    return sc_agg


def _tc_proj(h, wrel, wroot, b):
    n_pad, d = h.shape
    hd = wrel.shape[1]
    grid = n_pad // BN

    def body(h_ref, wrel_ref, wroot_ref, b_ref, hr_ref, hroot_ref):
        hb = h_ref[...]
        hr_ref[...] = jnp.dot(hb, wrel_ref[...],
                              preferred_element_type=jnp.float32)
        hroot_ref[...] = jnp.dot(hb, wroot_ref[...],
                                 preferred_element_type=jnp.float32) + b_ref[...]

    return pl.pallas_call(
        body,
        grid=(grid,),
        in_specs=[
            pl.BlockSpec((BN, d), lambda i: (i, 0)),
            pl.BlockSpec((d, hd), lambda i: (0, 0)),
            pl.BlockSpec((d, hd), lambda i: (0, 0)),
            pl.BlockSpec((1, hd), lambda i: (0, 0)),
        ],
        out_specs=[
            pl.BlockSpec((BN, hd), lambda i: (i, 0)),
            pl.BlockSpec((BN, hd), lambda i: (i, 0)),
        ],
        out_shape=[jax.ShapeDtypeStruct((n_pad, hd), jnp.float32)] * 2,
    )(h, wrel, wroot, b.reshape(1, hd))


def _tc_combine_proj(agg, root_prev, wrel, wroot, b):
    n2, hd = agg.shape
    n_pad = n2 // NC
    grid = n_pad // BN

    def body(a0_ref, a1_ref, root_ref, wrel_ref, wroot_ref, b_ref,
             hr_ref, hroot_ref):
        hb = jnp.maximum(a0_ref[...] + a1_ref[...] + root_ref[...], 0.0)
        hr_ref[...] = jnp.dot(hb, wrel_ref[...],
                              preferred_element_type=jnp.float32)
        hroot_ref[...] = jnp.dot(hb, wroot_ref[...],
                                 preferred_element_type=jnp.float32) + b_ref[...]

    return pl.pallas_call(
        body,
        grid=(grid,),
        in_specs=[
            pl.BlockSpec((BN, hd), lambda i: (i, 0)),
            pl.BlockSpec((BN, hd), lambda i: (i + grid, 0)),
            pl.BlockSpec((BN, hd), lambda i: (i, 0)),
            pl.BlockSpec((hd, hd), lambda i: (0, 0)),
            pl.BlockSpec((hd, hd), lambda i: (0, 0)),
            pl.BlockSpec((1, hd), lambda i: (0, 0)),
        ],
        out_specs=[
            pl.BlockSpec((BN, hd), lambda i: (i, 0)),
            pl.BlockSpec((BN, hd), lambda i: (i, 0)),
        ],
        out_shape=[jax.ShapeDtypeStruct((n_pad, hd), jnp.float32)] * 2,
    )(agg, agg, root_prev, wrel, wroot, b.reshape(1, hd))


def _tc_final(agg, root_prev, batch3d, wl, bl, g: int):
    n2, hd = agg.shape
    n_pad = n2 // NC
    grid = n_pad // BN
    c_dim = wl.shape[1]

    def body(a0_ref, a1_ref, root_ref, batch_ref, wl_ref, bl_ref,
             pooled_ref, out_ref, sums_ref, cnt_ref):
        i = pl.program_id(0)

        @pl.when(i == 0)
        def _init():
            sums_ref[...] = jnp.zeros_like(sums_ref)
            cnt_ref[...] = jnp.zeros_like(cnt_ref)

        h3 = a0_ref[...] + a1_ref[...] + root_ref[...]
        bvec = batch_ref[0]
        pt = (lax.broadcasted_iota(jnp.int32, (g, 1), 0) == bvec
              ).astype(jnp.float32)
        sums_ref[...] += jnp.dot(pt, h3, preferred_element_type=jnp.float32)
        cnt_ref[...] += jnp.broadcast_to(
            jnp.sum(pt, axis=1, keepdims=True), cnt_ref.shape)

        @pl.when(i == grid - 1)
        def _fin():
            pooled = sums_ref[...] / jnp.maximum(cnt_ref[...], 1.0)
            pooled_ref[...] = pooled
            out_ref[...] = jnp.dot(pooled, wl_ref[...],
                                   preferred_element_type=jnp.float32) + bl_ref[...]

    return pl.pallas_call(
        body,
        grid=(grid,),
        in_specs=[
            pl.BlockSpec((BN, hd), lambda i: (i, 0)),
            pl.BlockSpec((BN, hd), lambda i: (i + grid, 0)),
            pl.BlockSpec((BN, hd), lambda i: (i, 0)),
            pl.BlockSpec((1, 1, BN), lambda i: (i, 0, 0)),
            pl.BlockSpec((hd, c_dim), lambda i: (0, 0)),
            pl.BlockSpec((1, c_dim), lambda i: (0, 0)),
        ],
        out_specs=[
            pl.BlockSpec((g, hd), lambda i: (0, 0)),
            pl.BlockSpec((g, c_dim), lambda i: (0, 0)),
        ],
        out_shape=[
            jax.ShapeDtypeStruct((g, hd), jnp.float32),
            jax.ShapeDtypeStruct((g, c_dim), jnp.float32),
        ],
        scratch_shapes=[
            pltpu.VMEM((g, hd), jnp.float32),
            pltpu.VMEM((g, hd), jnp.float32),
        ],
    )(agg, agg, root_prev, batch3d, wl, bl.reshape(1, c_dim))


def kernel(x, edge_index, batch, W1_rel, W1_root, b1, W2_rel, W2_root, b2,
           W3_rel, W3_root, b3, Wl, bl):
    n, d = x.shape
    h_dim = W1_rel.shape[1]
    e = edge_index.shape[1]
    g = 64

    n_pad = ((n + BN) // BN) * BN
    ch = -(-e // (NW * LK))
    e_pad = NW * LK * ch

    x_p = jnp.zeros((n_pad, d), x.dtype).at[:n].set(x)
    pad_e = jnp.full((e_pad - e,), n, jnp.int32)
    src_p = jnp.concatenate([edge_index[0], pad_e]).reshape(NW, ch, LK)
    dst_p = jnp.concatenate([edge_index[1], pad_e]).reshape(NW, ch, LK)
    batch_p = jnp.concatenate(
        [batch.astype(jnp.int32), jnp.full((n_pad - n,), g, jnp.int32)]
    ).reshape(n_pad // BN, 1, BN)

    sc_agg = _make_sc_agg(n_pad, h_dim, ch)

    xr, xroot = _tc_proj(x_p, W1_rel, W1_root, b1)
    agg1 = sc_agg(xr, src_p, dst_p)
    h1r, h1root = _tc_combine_proj(agg1, xroot, W2_rel, W2_root, b2)
    agg2 = sc_agg(h1r, src_p, dst_p)
    h2r, h2root = _tc_combine_proj(agg2, h1root, W3_rel, W3_root, b3)
    agg3 = sc_agg(h2r, src_p, dst_p)
    pooled, out = _tc_final(agg3, h2root, batch_p, Wl, bl, g)
    return (pooled, out)
